# Initial kernel scaffold; baseline (speedup 1.0000x reference)
import functools

import jax
import jax.numpy as jnp
from jax.experimental import pallas as pl
from jax.experimental.pallas import tpu as pltpu
from jax.experimental.pallas import tpu_sc as plsc

_WINDOW = 80


def _gather_rows_sc(x2d, idx_all):
    num_idx = idx_all.shape[1]
    d = x2d.shape[1]
    mesh = plsc.VectorSubcoreMesh(core_axis_name="c", subcore_axis_name="s")

    @functools.partial(
        pl.kernel,
        out_type=jax.ShapeDtypeStruct((num_idx, d), x2d.dtype),
        mesh=mesh,
    )
    def gather_kernel(x_hbm, i_hbm, o_hbm):
        def body(i_vmem, o_vmem):
            pltpu.sync_copy(x_hbm.at[i_vmem.at[0]], o_vmem)

        pltpu.emit_pipeline(
            body,
            grid=(num_idx // _WINDOW,),
            in_specs=[pl.BlockSpec((1, _WINDOW), index_map=lambda i: (0, i))],
            out_specs=[pl.BlockSpec((_WINDOW, d), index_map=lambda i: (i, 0))],
            core_axis_name=("c", "s"),
            dimension_semantics=(pltpu.PARALLEL,),
        )(i_hbm, o_hbm)

    return gather_kernel(x2d, idx_all)


def kernel(x, indices):
    b, n, d = x.shape
    k = indices.shape[0]
    x2d = x.reshape(b * n, d)
    offsets = (jnp.arange(b, dtype=jnp.int32) * n)[:, None]
    idx_all = (indices[None, :] + offsets).reshape(1, b * k)
    out = _gather_rows_sc(x2d, idx_all)
    return out.reshape(b, k, d)


# speedup vs baseline: 1.3827x; 1.3827x over previous
import functools

import jax
import jax.numpy as jnp
from jax import lax
from jax.experimental import pallas as pl
from jax.experimental.pallas import tpu as pltpu
from jax.experimental.pallas import tpu_sc as plsc

_NC = 2
_NS = 16
_NW = _NC * _NS

_CHUNK = 80


def _gather_rows_sc(x2d, idx_all):
    num_idx = idx_all.shape[0]
    d = x2d.shape[1]
    nchunks = num_idx // _CHUNK
    per_worker = -(-nchunks // _NW)
    mesh = plsc.VectorSubcoreMesh(core_axis_name="c", subcore_axis_name="s")

    @functools.partial(
        pl.kernel,
        out_type=jax.ShapeDtypeStruct((num_idx, d), x2d.dtype),
        mesh=mesh,
        scratch_types=[
            pltpu.VMEM((_CHUNK,), jnp.int32),
            pltpu.VMEM((_CHUNK, d), x2d.dtype),
            pltpu.SemaphoreType.DMA,
        ],
    )
    def gather_kernel(x_hbm, i_hbm, o_hbm, idx_v, rows_v, sem):
        wid = lax.axis_index("s") * _NC + lax.axis_index("c")

        @pl.loop(0, per_worker)
        def _(i):
            c = i * _NW + wid

            @pl.when(c < nchunks)
            def _():
                base = c * _CHUNK
                pltpu.sync_copy(i_hbm.at[pl.ds(base, _CHUNK)], idx_v)
                pltpu.async_copy(x_hbm.at[idx_v], rows_v, sem).wait()
                pltpu.sync_copy(rows_v, o_hbm.at[pl.ds(base, _CHUNK)])

    return gather_kernel(x2d, idx_all)


def kernel(x, indices):
    b, n, d = x.shape
    k = indices.shape[0]
    x2d = x.reshape(b * n, d)
    offsets = (jnp.arange(b, dtype=jnp.int32) * n)[:, None]
    idx_all = (indices[None, :] + offsets).reshape(b * k)
    out = _gather_rows_sc(x2d, idx_all)
    return out.reshape(b, k, d)


# speedup vs baseline: 2.0965x; 1.5162x over previous
import functools

import jax
import jax.numpy as jnp
from jax import lax
from jax.experimental import pallas as pl
from jax.experimental.pallas import tpu as pltpu
from jax.experimental.pallas import tpu_sc as plsc

_NC = 2
_NS = 16
_NW = _NC * _NS

_CHUNK = 80
_NBUF = 4
_DEPTH = 2


def _gather_rows_sc(x2d, idx_all):
    num_idx = idx_all.shape[0]
    d = x2d.shape[1]
    g = _CHUNK
    nchunks = num_idx // g
    pc = nchunks // _NW
    rem = nchunks % _NW
    max_pc = pc + (1 if rem else 0)
    mesh = plsc.VectorSubcoreMesh(core_axis_name="c", subcore_axis_name="s")

    @functools.partial(
        pl.kernel,
        out_type=jax.ShapeDtypeStruct((num_idx, d), x2d.dtype),
        mesh=mesh,
        scratch_types=(
            [pltpu.VMEM((max_pc * g,), jnp.int32)]
            + [pltpu.VMEM((g, d), x2d.dtype) for _ in range(_NBUF)]
            + [pltpu.SemaphoreType.DMA for _ in range(2 * _NBUF)]
        ),
    )
    def gather_kernel(x_hbm, i_hbm, o_hbm, idx_v, *bufs_and_sems):
        rows = list(bufs_and_sems[:_NBUF])
        sem_g = list(bufs_and_sems[_NBUF:2 * _NBUF])
        sem_s = list(bufs_and_sems[2 * _NBUF:])

        wid = lax.axis_index("s") * _NC + lax.axis_index("c")
        start_chunk = wid * pc + jnp.minimum(wid, rem)
        has_extra = wid < rem
        row_base = start_chunk * g

        pltpu.sync_copy(i_hbm.at[pl.ds(row_base, pc * g)],
                        idx_v.at[pl.ds(0, pc * g)])

        @pl.when(has_extra)
        def _():
            pltpu.sync_copy(i_hbm.at[pl.ds(row_base + pc * g, g)],
                            idx_v.at[pl.ds(pc * g, g)])

        def valid(c):
            return (c < pc) | ((c < max_pc) & has_extra)

        def gather_copy(c, b):
            return pltpu.make_async_copy(
                x_hbm.at[idx_v.at[pl.ds(c * g, g)]], rows[b], sem_g[b])

        def store_copy(c, b):
            return pltpu.make_async_copy(
                rows[b], o_hbm.at[pl.ds(row_base + c * g, g)], sem_s[b])

        for b in range(_DEPTH):
            gather_copy(b, b).start()

        @pl.loop(0, max_pc, step=_NBUF)
        def _(outer):
            for k in range(_NBUF):
                j = outer + k
                bk = k
                b_ahead = (k + _DEPTH) % _NBUF

                @pl.when((j >= _DEPTH) & valid(j - _DEPTH))
                def _(j=j, b=b_ahead):
                    store_copy(j - _DEPTH, b).wait()

                @pl.when(valid(j + _DEPTH))
                def _(j=j, b=b_ahead):
                    gather_copy(j + _DEPTH, b).start()

                @pl.when(valid(j))
                def _(j=j, b=bk):
                    gather_copy(j, b).wait()
                    store_copy(j, b).start()

        store_copy(pc - 1, (pc - 1) % _NBUF).wait()

        @pl.when(has_extra)
        def _():
            store_copy(max_pc - 1, (max_pc - 1) % _NBUF).wait()

    return gather_kernel(x2d, idx_all)


def kernel(x, indices):
    b, n, d = x.shape
    k = indices.shape[0]
    x2d = x.reshape(b * n, d)
    offsets = (jnp.arange(b, dtype=jnp.int32) * n)[:, None]
    idx_all = (indices[None, :] + offsets).reshape(b * k)
    out = _gather_rows_sc(x2d, idx_all)
    return out.reshape(b, k, d)


# speedup vs baseline: 2.1420x; 1.0217x over previous
import functools

import jax
import jax.numpy as jnp
from jax import lax
from jax.experimental import pallas as pl
from jax.experimental.pallas import tpu as pltpu
from jax.experimental.pallas import tpu_sc as plsc

_NC = 2
_NS = 16
_NW = _NC * _NS

_CHUNK = 80
_NBUF = 6
_DEPTH = 3


def _gather_rows_sc(x2d, idx_all):
    num_idx = idx_all.shape[0]
    d = x2d.shape[1]
    g = _CHUNK
    nchunks = num_idx // g
    pc = nchunks // _NW
    rem = nchunks % _NW
    max_pc = pc + (1 if rem else 0)
    mesh = plsc.VectorSubcoreMesh(core_axis_name="c", subcore_axis_name="s")

    @functools.partial(
        pl.kernel,
        out_type=jax.ShapeDtypeStruct((num_idx, d), x2d.dtype),
        mesh=mesh,
        scratch_types=(
            [pltpu.VMEM((max_pc * g,), jnp.int32)]
            + [pltpu.VMEM((g, d), x2d.dtype) for _ in range(_NBUF)]
            + [pltpu.SemaphoreType.DMA for _ in range(2 * _NBUF)]
        ),
    )
    def gather_kernel(x_hbm, i_hbm, o_hbm, idx_v, *bufs_and_sems):
        rows = list(bufs_and_sems[:_NBUF])
        sem_g = list(bufs_and_sems[_NBUF:2 * _NBUF])
        sem_s = list(bufs_and_sems[2 * _NBUF:])

        wid = lax.axis_index("s") * _NC + lax.axis_index("c")
        start_chunk = wid * pc + jnp.minimum(wid, rem)
        has_extra = wid < rem
        row_base = start_chunk * g

        pltpu.sync_copy(i_hbm.at[pl.ds(row_base, pc * g)],
                        idx_v.at[pl.ds(0, pc * g)])

        @pl.when(has_extra)
        def _():
            pltpu.sync_copy(i_hbm.at[pl.ds(row_base + pc * g, g)],
                            idx_v.at[pl.ds(pc * g, g)])

        def valid(c):
            return (c < pc) | ((c < max_pc) & has_extra)

        def gather_copy(c, b):
            return pltpu.make_async_copy(
                x_hbm.at[idx_v.at[pl.ds(c * g, g)]], rows[b], sem_g[b])

        def store_copy(c, b):
            return pltpu.make_async_copy(
                rows[b], o_hbm.at[pl.ds(row_base + c * g, g)], sem_s[b])

        for b in range(_DEPTH):
            gather_copy(b, b).start()

        @pl.loop(0, max_pc, step=_NBUF)
        def _(outer):
            for k in range(_NBUF):
                j = outer + k
                bk = k
                b_ahead = (k + _DEPTH) % _NBUF

                @pl.when((j >= _DEPTH) & valid(j - _DEPTH))
                def _(j=j, b=b_ahead):
                    store_copy(j - _DEPTH, b).wait()

                @pl.when(valid(j + _DEPTH))
                def _(j=j, b=b_ahead):
                    gather_copy(j + _DEPTH, b).start()

                @pl.when(valid(j))
                def _(j=j, b=bk):
                    gather_copy(j, b).wait()
                    store_copy(j, b).start()

        covered = -(-max_pc // _NBUF) * _NBUF
        for c in range(covered - _DEPTH, max_pc):
            @pl.when(valid(c))
            def _(c=c):
                store_copy(c, c % _NBUF).wait()

    return gather_kernel(x2d, idx_all)


def kernel(x, indices):
    b, n, d = x.shape
    k = indices.shape[0]
    x2d = x.reshape(b * n, d)
    offsets = (jnp.arange(b, dtype=jnp.int32) * n)[:, None]
    idx_all = (indices[None, :] + offsets).reshape(b * k)
    out = _gather_rows_sc(x2d, idx_all)
    return out.reshape(b, k, d)


# speedup vs baseline: 2.1572x; 1.0071x over previous
import functools

import jax
import jax.numpy as jnp
from jax import lax
from jax.experimental import pallas as pl
from jax.experimental.pallas import tpu as pltpu
from jax.experimental.pallas import tpu_sc as plsc

_NC = 2
_NS = 16
_NW = _NC * _NS

_CHUNK = 80
_NBUF = 8
_DEPTH = 4


def _gather_rows_sc(x2d, idx_all):
    num_idx = idx_all.shape[0]
    d = x2d.shape[1]
    g = _CHUNK
    nchunks = num_idx // g
    pc = nchunks // _NW
    rem = nchunks % _NW
    max_pc = pc + (1 if rem else 0)
    mesh = plsc.VectorSubcoreMesh(core_axis_name="c", subcore_axis_name="s")

    @functools.partial(
        pl.kernel,
        out_type=jax.ShapeDtypeStruct((num_idx, d), x2d.dtype),
        mesh=mesh,
        scratch_types=(
            [pltpu.VMEM((max_pc * g,), jnp.int32)]
            + [pltpu.VMEM((g, d), x2d.dtype) for _ in range(_NBUF)]
            + [pltpu.SemaphoreType.DMA for _ in range(2 * _NBUF)]
        ),
    )
    def gather_kernel(x_hbm, i_hbm, o_hbm, idx_v, *bufs_and_sems):
        rows = list(bufs_and_sems[:_NBUF])
        sem_g = list(bufs_and_sems[_NBUF:2 * _NBUF])
        sem_s = list(bufs_and_sems[2 * _NBUF:])

        wid = lax.axis_index("s") * _NC + lax.axis_index("c")
        start_chunk = wid * pc + jnp.minimum(wid, rem)
        has_extra = wid < rem
        row_base = start_chunk * g

        pltpu.sync_copy(i_hbm.at[pl.ds(row_base, pc * g)],
                        idx_v.at[pl.ds(0, pc * g)])

        @pl.when(has_extra)
        def _():
            pltpu.sync_copy(i_hbm.at[pl.ds(row_base + pc * g, g)],
                            idx_v.at[pl.ds(pc * g, g)])

        def valid(c):
            return (c < pc) | ((c < max_pc) & has_extra)

        def gather_copy(c, b):
            return pltpu.make_async_copy(
                x_hbm.at[idx_v.at[pl.ds(c * g, g)]], rows[b], sem_g[b])

        def store_copy(c, b):
            return pltpu.make_async_copy(
                rows[b], o_hbm.at[pl.ds(row_base + c * g, g)], sem_s[b])

        for b in range(_DEPTH):
            gather_copy(b, b).start()

        @pl.loop(0, max_pc, step=_NBUF)
        def _(outer):
            for k in range(_NBUF):
                j = outer + k
                bk = k
                b_ahead = (k + _DEPTH) % _NBUF

                @pl.when((j >= _DEPTH) & valid(j - _DEPTH))
                def _(j=j, b=b_ahead):
                    store_copy(j - _DEPTH, b).wait()

                @pl.when(valid(j + _DEPTH))
                def _(j=j, b=b_ahead):
                    gather_copy(j + _DEPTH, b).start()

                @pl.when(valid(j))
                def _(j=j, b=bk):
                    gather_copy(j, b).wait()
                    store_copy(j, b).start()

        covered = -(-max_pc // _NBUF) * _NBUF
        for c in range(covered - _DEPTH, max_pc):
            @pl.when(valid(c))
            def _(c=c):
                store_copy(c, c % _NBUF).wait()

    return gather_kernel(x2d, idx_all)


def kernel(x, indices):
    b, n, d = x.shape
    k = indices.shape[0]
    x2d = x.reshape(b * n, d)
    offsets = (jnp.arange(b, dtype=jnp.int32) * n)[:, None]
    idx_all = (indices[None, :] + offsets).reshape(b * k)
    out = _gather_rows_sc(x2d, idx_all)
    return out.reshape(b, k, d)


# speedup vs baseline: 2.1613x; 1.0019x over previous
import functools

import jax
import jax.numpy as jnp
from jax import lax
from jax.experimental import pallas as pl
from jax.experimental.pallas import tpu as pltpu
from jax.experimental.pallas import tpu_sc as plsc

_NC = 2
_NS = 16
_NW = _NC * _NS

_CHUNK = 80
_NBUF = 12
_DEPTH = 6


def _gather_rows_sc(x2d, idx_all):
    num_idx = idx_all.shape[0]
    d = x2d.shape[1]
    g = _CHUNK
    nchunks = num_idx // g
    pc = nchunks // _NW
    rem = nchunks % _NW
    max_pc = pc + (1 if rem else 0)
    mesh = plsc.VectorSubcoreMesh(core_axis_name="c", subcore_axis_name="s")

    @functools.partial(
        pl.kernel,
        out_type=jax.ShapeDtypeStruct((num_idx, d), x2d.dtype),
        mesh=mesh,
        scratch_types=(
            [pltpu.VMEM((max_pc * g,), jnp.int32)]
            + [pltpu.VMEM((g, d), x2d.dtype) for _ in range(_NBUF)]
            + [pltpu.SemaphoreType.DMA for _ in range(2 * _NBUF)]
        ),
    )
    def gather_kernel(x_hbm, i_hbm, o_hbm, idx_v, *bufs_and_sems):
        rows = list(bufs_and_sems[:_NBUF])
        sem_g = list(bufs_and_sems[_NBUF:2 * _NBUF])
        sem_s = list(bufs_and_sems[2 * _NBUF:])

        wid = lax.axis_index("s") * _NC + lax.axis_index("c")
        start_chunk = wid * pc + jnp.minimum(wid, rem)
        has_extra = wid < rem
        row_base = start_chunk * g

        pltpu.sync_copy(i_hbm.at[pl.ds(row_base, pc * g)],
                        idx_v.at[pl.ds(0, pc * g)])

        @pl.when(has_extra)
        def _():
            pltpu.sync_copy(i_hbm.at[pl.ds(row_base + pc * g, g)],
                            idx_v.at[pl.ds(pc * g, g)])

        def valid(c):
            return (c < pc) | ((c < max_pc) & has_extra)

        def gather_copy(c, b):
            return pltpu.make_async_copy(
                x_hbm.at[idx_v.at[pl.ds(c * g, g)]], rows[b], sem_g[b])

        def store_copy(c, b):
            return pltpu.make_async_copy(
                rows[b], o_hbm.at[pl.ds(row_base + c * g, g)], sem_s[b])

        for b in range(_DEPTH):
            gather_copy(b, b).start()

        @pl.loop(0, max_pc, step=_NBUF)
        def _(outer):
            for k in range(_NBUF):
                j = outer + k
                bk = k
                b_ahead = (k + _DEPTH) % _NBUF

                @pl.when((j >= _DEPTH) & valid(j - _DEPTH))
                def _(j=j, b=b_ahead):
                    store_copy(j - _DEPTH, b).wait()

                @pl.when(valid(j + _DEPTH))
                def _(j=j, b=b_ahead):
                    gather_copy(j + _DEPTH, b).start()

                @pl.when(valid(j))
                def _(j=j, b=bk):
                    gather_copy(j, b).wait()
                    store_copy(j, b).start()

        covered = -(-max_pc // _NBUF) * _NBUF
        for c in range(covered - _DEPTH, max_pc):
            @pl.when(valid(c))
            def _(c=c):
                store_copy(c, c % _NBUF).wait()

    return gather_kernel(x2d, idx_all)


def kernel(x, indices):
    b, n, d = x.shape
    k = indices.shape[0]
    x2d = x.reshape(b * n, d)
    offsets = (jnp.arange(b, dtype=jnp.int32) * n)[:, None]
    idx_all = (indices[None, :] + offsets).reshape(b * k)
    out = _gather_rows_sc(x2d, idx_all)
    return out.reshape(b, k, d)
